# baseline (speedup 1.0000x reference)
import jax
import jax.numpy as jnp
from jax.experimental import pallas as pl
from jax.experimental.pallas import tpu as pltpu


def _round_up(x, m):
    return ((x + m - 1) // m) * m


def _pick_tn(vocab, target):
    target = max(128, (target // 128) * 128)
    if vocab <= target:
        return vocab, vocab
    if vocab % 128 == 0:
        cand = target
        while cand >= 128:
            if vocab % cand == 0:
                return cand, vocab
            cand -= 128
    return target, _round_up(vocab, target)


def _fused_kernel(x_ref, wd_ref, wo_ref, o_ref, h_ref):
    @pl.when(pl.program_id(1) == 0)
    def _():
        h_ref[...] = jnp.dot(
            x_ref[...].astype(jnp.bfloat16),
            wd_ref[...].astype(jnp.bfloat16),
            preferred_element_type=jnp.float32,
        ).astype(jnp.bfloat16)

    o_ref[...] = jnp.dot(
        h_ref[...],
        wo_ref[...].astype(jnp.bfloat16),
        preferred_element_type=jnp.float32,
    ).astype(o_ref.dtype)


def kernel(x, w_dense, w_out):
    batch, seq, hidden = x.shape
    bottleneck = w_dense.shape[1]
    vocab = w_out.shape[1]
    M = batch * seq
    x2d = x.reshape(M, hidden)

    TM = min(2048, _round_up(M, 8))
    m_tiles = pl.cdiv(M, TM)
    m_pad = m_tiles * TM
    if m_pad != M:
        x2d = jnp.pad(x2d, ((0, m_pad - M), (0, 0)))

    TN, v_pad = _pick_tn(vocab, 640)
    w_out_p = w_out if v_pad == vocab else jnp.pad(
        w_out, ((0, 0), (0, v_pad - vocab)))

    grid = (m_tiles, v_pad // TN)
    out_is = jnp.dtype(x.dtype).itemsize
    cost = pl.CostEstimate(
        flops=int(2 * M * hidden * bottleneck + 2 * M * bottleneck * vocab),
        transcendentals=0,
        bytes_accessed=int(
            m_pad * hidden * jnp.dtype(x.dtype).itemsize
            + hidden * bottleneck * jnp.dtype(w_dense.dtype).itemsize
            + m_tiles * bottleneck * v_pad * jnp.dtype(w_out.dtype).itemsize
            + m_pad * v_pad * out_is
        ),
    )

    out2d = pl.pallas_call(
        _fused_kernel,
        out_shape=jax.ShapeDtypeStruct((m_pad, v_pad), x.dtype),
        grid=grid,
        in_specs=[
            pl.BlockSpec((TM, hidden), lambda i, j: (i, 0)),
            pl.BlockSpec((hidden, bottleneck), lambda i, j: (0, 0)),
            pl.BlockSpec((bottleneck, TN), lambda i, j: (0, j)),
        ],
        out_specs=pl.BlockSpec((TM, TN), lambda i, j: (i, j)),
        scratch_shapes=[pltpu.VMEM((TM, bottleneck), jnp.bfloat16)],
        compiler_params=pltpu.CompilerParams(
            dimension_semantics=("parallel", "arbitrary"),
            vmem_limit_bytes=60 * 1024 * 1024,
        ),
        cost_estimate=cost,
    )(x2d, w_dense, w_out_p)

    out2d = out2d[:M, :vocab] if (m_pad != M or v_pad != vocab) else out2d
    return out2d.reshape(batch, seq, vocab)


# speedup vs baseline: 1.1472x; 1.1472x over previous
import jax
import jax.numpy as jnp
from jax.experimental import pallas as pl
from jax.experimental.pallas import tpu as pltpu


def _round_up(x, m):
    return ((x + m - 1) // m) * m


def _pick_tn(vocab, target):
    target = max(128, (target // 128) * 128)
    if vocab <= target:
        return vocab, vocab
    if vocab % 128 == 0:
        cand = target
        while cand >= 128:
            if vocab % cand == 0:
                return cand, vocab
            cand -= 128
    return target, _round_up(vocab, target)


def _bottleneck_kernel(x_ref, wd_ref, h_ref):
    h_ref[...] = jnp.dot(
        x_ref[...].astype(jnp.bfloat16),
        wd_ref[...].astype(jnp.bfloat16),
        preferred_element_type=jnp.float32,
    ).astype(jnp.bfloat16)


def _vocab_proj_kernel(h_ref, wo_ref, o_ref):
    o_ref[...] = jnp.dot(
        h_ref[...],
        wo_ref[...].astype(jnp.bfloat16),
        preferred_element_type=jnp.float32,
    ).astype(o_ref.dtype)


def kernel(x, w_dense, w_out):
    batch, seq, hidden = x.shape
    bottleneck = w_dense.shape[1]
    vocab = w_out.shape[1]
    M = batch * seq
    x2d = x.reshape(M, hidden)

    TM = min(2048, _round_up(M, 16))
    m_tiles = pl.cdiv(M, TM)
    m_pad = m_tiles * TM
    if m_pad != M:
        x2d = jnp.pad(x2d, ((0, m_pad - M), (0, 0)))

    h = pl.pallas_call(
        _bottleneck_kernel,
        out_shape=jax.ShapeDtypeStruct((m_pad, bottleneck), jnp.bfloat16),
        grid=(m_tiles,),
        in_specs=[
            pl.BlockSpec((TM, hidden), lambda i: (i, 0)),
            pl.BlockSpec((hidden, bottleneck), lambda i: (0, 0)),
        ],
        out_specs=pl.BlockSpec((TM, bottleneck), lambda i: (i, 0)),
        compiler_params=pltpu.CompilerParams(
            dimension_semantics=("parallel",),
        ),
        cost_estimate=pl.CostEstimate(
            flops=int(2 * m_pad * hidden * bottleneck),
            transcendentals=0,
            bytes_accessed=int(m_pad * hidden * 4 + hidden * bottleneck * 4
                               + m_pad * bottleneck * 2),
        ),
    )(x2d, w_dense)

    TN, v_pad = _pick_tn(vocab, 640)
    w_out_p = w_out if v_pad == vocab else jnp.pad(
        w_out, ((0, 0), (0, v_pad - vocab)))

    out2d = pl.pallas_call(
        _vocab_proj_kernel,
        out_shape=jax.ShapeDtypeStruct((m_pad, v_pad), x.dtype),
        grid=(v_pad // TN,),
        in_specs=[
            pl.BlockSpec((m_pad, bottleneck), lambda j: (0, 0)),
            pl.BlockSpec((bottleneck, TN), lambda j: (0, j)),
        ],
        out_specs=pl.BlockSpec((m_pad, TN), lambda j: (0, j)),
        compiler_params=pltpu.CompilerParams(
            dimension_semantics=("parallel",),
            vmem_limit_bytes=58 * 1024 * 1024,
        ),
        cost_estimate=pl.CostEstimate(
            flops=int(2 * m_pad * bottleneck * v_pad),
            transcendentals=0,
            bytes_accessed=int(m_pad * bottleneck * 2
                               + bottleneck * v_pad * 4
                               + m_pad * v_pad * 4),
        ),
    )(h, w_out_p)

    out2d = out2d[:M, :vocab] if (m_pad != M or v_pad != vocab) else out2d
    return out2d.reshape(batch, seq, vocab)
